# baseline (speedup 1.0000x reference)
import functools

import jax
import jax.numpy as jnp
from jax import lax
from jax.experimental import pallas as pl
from jax.experimental.pallas import tpu as pltpu
from jax.experimental.pallas import tpu_sc as plsc

N = 10000
D = 128
E = 320000

NC = 2
NS = 16
NW = NC * NS
BD = 32
BE = 80
EP = 327680
RD = EP // NW // BD
NP = 10240
RT = NP // NS

_mesh = plsc.VectorSubcoreMesh(
    core_axis_name="c", subcore_axis_name="s", num_cores=NC, num_subcores=NS)


@functools.partial(
    pl.kernel,
    out_type=jax.ShapeDtypeStruct((2 * NP, D), jnp.float32),
    mesh=_mesh,
    scratch_types=[
        pltpu.VMEM((BD,), jnp.int32),
        pltpu.VMEM((BD,), jnp.int32),
        pltpu.VMEM((BD, D), jnp.float32),
        pltpu.VMEM_SHARED((NP, D), jnp.float32),
        pltpu.SemaphoreType.DMA,
        pltpu.SemaphoreType.DMA,
        pltpu.SemaphoreType.DMA,
        pltpu.SemaphoreType.DMA,
    ],
)
def _deg_kernel(dst, zerosf, onesf, out,
                id0, id1, ones_v, acc, di0, di1, ss0, ss1):
    cid = lax.axis_index("c")
    sid = lax.axis_index("s")
    wid = sid * NC + cid
    r0 = sid * RT
    e0 = wid * RD * BD
    pltpu.sync_copy(zerosf.at[pl.ds(r0, RT)], acc.at[pl.ds(r0, RT)])
    pltpu.sync_copy(onesf, ones_v)
    plsc.subcore_barrier()

    pltpu.async_copy(dst.at[pl.ds(e0, BD)], id0, di0)
    pltpu.async_copy(dst.at[pl.ds(e0 + BD, BD)], id1, di1)

    def body(i, carry):
        j0 = 2 * i
        j1 = 2 * i + 1
        pltpu.make_async_copy(dst.at[pl.ds(e0, BD)], id0, di0).wait()
        pltpu.async_copy(ones_v, acc.at[id0], ss0, add=True)
        pltpu.make_async_copy(dst.at[pl.ds(e0, BD)], id1, di1).wait()
        pltpu.async_copy(ones_v, acc.at[id1], ss1, add=True)
        pltpu.make_async_copy(ones_v, acc.at[id0], ss0).wait()

        @pl.when(j0 + 2 < RD)
        def _s0():
            pltpu.async_copy(dst.at[pl.ds(e0 + (j0 + 2) * BD, BD)], id0, di0)

        pltpu.make_async_copy(ones_v, acc.at[id1], ss1).wait()

        @pl.when(j1 + 2 < RD)
        def _s1():
            pltpu.async_copy(dst.at[pl.ds(e0 + (j1 + 2) * BD, BD)], id1, di1)

        return carry

    lax.fori_loop(0, RD // 2, body, 0)
    plsc.subcore_barrier()
    pltpu.sync_copy(acc.at[pl.ds(r0, RT)], out.at[pl.ds(cid * NP + r0, RT)])


QN = NP // 4
QT = QN // NS
C2 = EP // NS // BE


@functools.partial(
    pl.kernel,
    out_type=jax.ShapeDtypeStruct((NP, D), jnp.float32),
    mesh=_mesh,
    scratch_types=[
        [pltpu.VMEM((BE,), jnp.int32)] * 4,
        [pltpu.VMEM((BE,), jnp.int32)] * 4,
        [pltpu.VMEM((BE, D), jnp.float32)] * 2,
        pltpu.VMEM_SHARED((NP, D), jnp.float32),
        pltpu.VMEM_SHARED((QN, D), jnp.float32),
        [pltpu.SemaphoreType.DMA] * 4,
        [pltpu.SemaphoreType.DMA] * 4,
        [pltpu.SemaphoreType.DMA] * 2,
        [pltpu.SemaphoreType.DMA] * 2,
    ],
)
def _edge_kernel(table, src, dst, zerosf, out,
                 isv, idv, m, tabs, acc, si, di, gs, ss):
    cid = lax.axis_index("c")
    sid = lax.axis_index("s")
    r0 = sid * RT
    e0 = sid * (EP // NS)
    pltpu.sync_copy(table.at[pl.ds(r0, RT)], tabs.at[pl.ds(r0, RT)])

    def stage(j, k):
        pltpu.async_copy(src.at[pl.ds(e0 + j * BE, BE)], isv[k], si[k])
        pltpu.async_copy(dst.at[pl.ds(e0 + j * BE, BE)], idv[k], di[k])

    def wait_stage(k):
        pltpu.make_async_copy(src.at[pl.ds(e0, BE)], isv[k], si[k]).wait()
        pltpu.make_async_copy(dst.at[pl.ds(e0, BE)], idv[k], di[k]).wait()

    for p in range(2):
        base = (2 * p + cid) * QN
        pltpu.sync_copy(zerosf.at[pl.ds(sid * QT, QT)],
                        acc.at[pl.ds(sid * QT, QT)])
        plsc.subcore_barrier()

        def mask(k, base=base):
            for t in range(BE // 16):
                sl = pl.ds(t * 16, 16)
                s16 = isv[k][sl]
                d16 = idv[k][sl]
                keep = (d16 >= base) & (d16 < base + QN)
                isv[k][sl] = jnp.where(keep, s16, -1)
                idv[k][sl] = jnp.where(keep, d16 - base, -1)

        stage(0, 0)
        stage(1, 1)

        def quad(i, carry, base=base):
            for u in range(4):
                j = 4 * i + u
                a = u % 2
                k = u
                kf = (u + 2) % 4

                @pl.when(j >= 2)
                def _drain():
                    pltpu.make_async_copy(m[a], acc.at[
                        plsc.Indices(idv[kf], ignored_value=-1)], ss[a]).wait()

                @pl.when(j + 2 < C2)
                def _restage():
                    stage(j + 2, kf)

                wait_stage(k)
                mask(k)
                pltpu.async_copy(
                    tabs.at[plsc.Indices(isv[k], ignored_value=-1)], m[a],
                    gs[a])
                pltpu.make_async_copy(
                    tabs.at[plsc.Indices(isv[k], ignored_value=-1)], m[a],
                    gs[a]).wait()
                pltpu.async_copy(
                    m[a], acc.at[plsc.Indices(idv[k], ignored_value=-1)],
                    ss[a], add=True)
            return carry

        lax.fori_loop(0, C2 // 4, quad, 0)
        pltpu.make_async_copy(m[0], acc.at[
            plsc.Indices(idv[2], ignored_value=-1)], ss[0]).wait()
        pltpu.make_async_copy(m[1], acc.at[
            plsc.Indices(idv[3], ignored_value=-1)], ss[1]).wait()
        plsc.subcore_barrier()
        pltpu.sync_copy(acc.at[pl.ds(sid * QT, QT)],
                        out.at[pl.ds(base + sid * QT, QT)])


BM = 640


def _dinv(d0_ref, d1_ref):
    deg = d0_ref[:, 0:1] + d1_ref[:, 0:1] + 1.0
    return lax.rsqrt(deg)


def _tc1_body(x_ref, w1_ref, d0_ref, d1_ref, t1_ref):
    t1_ref[:, :] = _dinv(d0_ref, d1_ref) * jnp.dot(
        x_ref[:, :], w1_ref[:, :], preferred_element_type=jnp.float32)


def _tc2_body(s_ref, t1_ref, d0_ref, d1_ref, b1_ref, w2_ref, t2_ref):
    dinv = _dinv(d0_ref, d1_ref)
    h1 = jnp.maximum(
        dinv * (s_ref[:, :] + t1_ref[:, :]) + b1_ref[:, :], 0.0)
    t2_ref[:, :] = dinv * jnp.dot(
        h1, w2_ref[:, :], preferred_element_type=jnp.float32)


def _tc3_body(s_ref, t2_ref, d0_ref, d1_ref, b2_ref, wf_ref, bf_ref,
              x_ref, ws_ref, bs_ref, out_ref):
    dinv = _dinv(d0_ref, d1_ref)
    h2 = jnp.maximum(
        dinv * (s_ref[:, :] + t2_ref[:, :]) + b2_ref[:, :], 0.0)
    out_ref[:, :] = (
        jnp.dot(h2, wf_ref[:, :], preferred_element_type=jnp.float32)
        + bf_ref[:, :]
        + jnp.dot(x_ref[:, :], ws_ref[:, :], preferred_element_type=jnp.float32)
        + bs_ref[:, :])


_feat_spec = pl.BlockSpec((BM, D), lambda i: (i, 0))
_deg_spec = pl.BlockSpec((BM, D), lambda i: (i, 0))
_w_spec = pl.BlockSpec((D, D), lambda i: (0, 0))
_b_spec = pl.BlockSpec((1, D), lambda i: (0, 0))
_GRID = (NP // BM,)
_OUT_F32 = jax.ShapeDtypeStruct((NP, D), jnp.float32)

_tc1 = pl.pallas_call(
    _tc1_body, grid=_GRID,
    in_specs=[_feat_spec, _w_spec, _deg_spec, _deg_spec],
    out_specs=_feat_spec, out_shape=_OUT_F32)

_tc2 = pl.pallas_call(
    _tc2_body, grid=_GRID,
    in_specs=[_feat_spec, _feat_spec, _deg_spec, _deg_spec,
              _b_spec, _w_spec],
    out_specs=_feat_spec, out_shape=_OUT_F32)

_tc3 = pl.pallas_call(
    _tc3_body, grid=_GRID,
    in_specs=[_feat_spec, _feat_spec, _deg_spec, _deg_spec,
              _b_spec, _w_spec, _b_spec, _feat_spec, _w_spec, _b_spec],
    out_specs=_feat_spec, out_shape=_OUT_F32)


def kernel(x, edge_index, W1, b1, W2, b2, W_skip, b_skip, W_final, b_final):
    f32 = jnp.float32
    src = edge_index[0].astype(jnp.int32)
    dst = edge_index[1].astype(jnp.int32)
    pad = EP - E
    src = jnp.concatenate([src, jnp.full((pad,), N, jnp.int32)])
    dst = jnp.concatenate([dst, jnp.full((pad,), N, jnp.int32)])
    xp = jnp.zeros((NP, D), f32).at[:N, :].set(x)
    onesf = jnp.ones((BD, D), f32)
    zerosf = jnp.zeros((NP, D), f32)
    b1r = b1.reshape(1, D)
    b2r = b2.reshape(1, D)
    bfr = b_final.reshape(1, D)
    bsr = b_skip.reshape(1, D)

    dd = _deg_kernel(dst, zerosf, onesf)
    d0, d1 = dd[:NP], dd[NP:]
    t1 = _tc1(xp, W1, d0, d1)
    s1 = _edge_kernel(t1, src, dst, zerosf)
    t2 = _tc2(s1, t1, d0, d1, b1r, W2)
    s2 = _edge_kernel(t2, src, dst, zerosf)
    outp = _tc3(s2, t2, d0, d1, b2r, W_final, bfr, xp, W_skip, bsr)
    return outp[:N]


# speedup vs baseline: 1.0143x; 1.0143x over previous
import functools

import jax
import jax.numpy as jnp
from jax import lax
from jax.experimental import pallas as pl
from jax.experimental.pallas import tpu as pltpu
from jax.experimental.pallas import tpu_sc as plsc

N = 10000
D = 128
E = 320000

NC = 2
NS = 16
NW = NC * NS
B = 64
R = 160
EP = NW * R * B
NP = 10240
RT = NP // NS

_mesh = plsc.VectorSubcoreMesh(
    core_axis_name="c", subcore_axis_name="s", num_cores=NC, num_subcores=NS)


@functools.partial(
    pl.kernel,
    out_type=jax.ShapeDtypeStruct((2 * NP, D), jnp.float32),
    mesh=_mesh,
    scratch_types=[
        pltpu.VMEM((B,), jnp.int32),
        pltpu.VMEM((B,), jnp.int32),
        pltpu.VMEM((B, D), jnp.float32),
        pltpu.VMEM_SHARED((NP, D), jnp.float32),
        pltpu.SemaphoreType.DMA,
        pltpu.SemaphoreType.DMA,
        pltpu.SemaphoreType.DMA,
        pltpu.SemaphoreType.DMA,
    ],
)
def _deg_kernel(dst, zerosf, onesf, out,
                id0, id1, ones_v, acc, di0, di1, ss0, ss1):
    cid = lax.axis_index("c")
    sid = lax.axis_index("s")
    wid = sid * NC + cid
    r0 = sid * RT
    e0 = wid * R * B
    pltpu.sync_copy(zerosf.at[pl.ds(r0, RT)], acc.at[pl.ds(r0, RT)])
    pltpu.sync_copy(onesf, ones_v)
    plsc.subcore_barrier()

    pltpu.async_copy(dst.at[pl.ds(e0, B)], id0, di0)
    pltpu.async_copy(dst.at[pl.ds(e0 + B, B)], id1, di1)

    def body(i, carry):
        j0 = 2 * i
        j1 = 2 * i + 1
        pltpu.make_async_copy(dst.at[pl.ds(e0, B)], id0, di0).wait()
        pltpu.async_copy(ones_v, acc.at[id0], ss0, add=True)
        pltpu.make_async_copy(dst.at[pl.ds(e0, B)], id1, di1).wait()
        pltpu.async_copy(ones_v, acc.at[id1], ss1, add=True)
        pltpu.make_async_copy(ones_v, acc.at[id0], ss0).wait()

        @pl.when(j0 + 2 < R)
        def _s0():
            pltpu.async_copy(dst.at[pl.ds(e0 + (j0 + 2) * B, B)], id0, di0)

        pltpu.make_async_copy(ones_v, acc.at[id1], ss1).wait()

        @pl.when(j1 + 2 < R)
        def _s1():
            pltpu.async_copy(dst.at[pl.ds(e0 + (j1 + 2) * B, B)], id1, di1)

        return carry

    lax.fori_loop(0, R // 2, body, 0)
    plsc.subcore_barrier()
    pltpu.sync_copy(acc.at[pl.ds(r0, RT)], out.at[pl.ds(cid * NP + r0, RT)])


QN = NP // 4
QT = QN // NS
C2 = EP // NS // B


@functools.partial(
    pl.kernel,
    out_type=jax.ShapeDtypeStruct((NP, D), jnp.float32),
    mesh=_mesh,
    scratch_types=[
        [pltpu.VMEM((B,), jnp.int32)] * 4,
        [pltpu.VMEM((B,), jnp.int32)] * 4,
        [pltpu.VMEM((B, D), jnp.float32)] * 2,
        pltpu.VMEM_SHARED((NP, D), jnp.float32),
        pltpu.VMEM_SHARED((QN, D), jnp.float32),
        [pltpu.SemaphoreType.DMA] * 4,
        [pltpu.SemaphoreType.DMA] * 4,
        [pltpu.SemaphoreType.DMA] * 2,
        [pltpu.SemaphoreType.DMA] * 2,
    ],
)
def _edge_kernel(table, src, dst, zerosf, out,
                 isv, idv, m, tabs, acc, si, di, gs, ss):
    cid = lax.axis_index("c")
    sid = lax.axis_index("s")
    r0 = sid * RT
    e0 = sid * (EP // NS)
    pltpu.sync_copy(table.at[pl.ds(r0, RT)], tabs.at[pl.ds(r0, RT)])

    def stage(j, k):
        pltpu.async_copy(src.at[pl.ds(e0 + j * B, B)], isv[k], si[k])
        pltpu.async_copy(dst.at[pl.ds(e0 + j * B, B)], idv[k], di[k])

    def wait_stage(k):
        pltpu.make_async_copy(src.at[pl.ds(e0, B)], isv[k], si[k]).wait()
        pltpu.make_async_copy(dst.at[pl.ds(e0, B)], idv[k], di[k]).wait()

    for p in range(2):
        base = (2 * p + cid) * QN
        pltpu.sync_copy(zerosf.at[pl.ds(sid * QT, QT)],
                        acc.at[pl.ds(sid * QT, QT)])
        plsc.subcore_barrier()

        def mask(k, base=base):
            for t in range(B // 16):
                sl = pl.ds(t * 16, 16)
                s16 = isv[k][sl]
                d16 = idv[k][sl]
                keep = (d16 >= base) & (d16 < base + QN)
                isv[k][sl] = jnp.where(keep, s16, -1)
                idv[k][sl] = jnp.where(keep, d16 - base, -1)

        stage(0, 0)
        stage(1, 1)

        def quad(i, carry, base=base):
            for u in range(4):
                j = 4 * i + u
                a = u % 2
                k = u
                kf = (u + 2) % 4

                @pl.when(j >= 2)
                def _drain():
                    pltpu.make_async_copy(m[a], acc.at[
                        plsc.Indices(idv[kf], ignored_value=-1)], ss[a]).wait()

                @pl.when(j + 2 < C2)
                def _restage():
                    stage(j + 2, kf)

                wait_stage(k)
                mask(k)
                pltpu.async_copy(
                    tabs.at[plsc.Indices(isv[k], ignored_value=-1)], m[a],
                    gs[a])
                pltpu.make_async_copy(
                    tabs.at[plsc.Indices(isv[k], ignored_value=-1)], m[a],
                    gs[a]).wait()
                pltpu.async_copy(
                    m[a], acc.at[plsc.Indices(idv[k], ignored_value=-1)],
                    ss[a], add=True)
            return carry

        lax.fori_loop(0, C2 // 4, quad, 0)
        pltpu.make_async_copy(m[0], acc.at[
            plsc.Indices(idv[2], ignored_value=-1)], ss[0]).wait()
        pltpu.make_async_copy(m[1], acc.at[
            plsc.Indices(idv[3], ignored_value=-1)], ss[1]).wait()
        plsc.subcore_barrier()
        pltpu.sync_copy(acc.at[pl.ds(sid * QT, QT)],
                        out.at[pl.ds(base + sid * QT, QT)])


BM = 640


def _dinv(d0_ref, d1_ref):
    deg = d0_ref[:, 0:1] + d1_ref[:, 0:1] + 1.0
    return lax.rsqrt(deg)


def _tc1_body(x_ref, w1_ref, d0_ref, d1_ref, t1_ref):
    t1_ref[:, :] = _dinv(d0_ref, d1_ref) * jnp.dot(
        x_ref[:, :], w1_ref[:, :], preferred_element_type=jnp.float32)


def _tc2_body(s_ref, t1_ref, d0_ref, d1_ref, b1_ref, w2_ref, t2_ref):
    dinv = _dinv(d0_ref, d1_ref)
    h1 = jnp.maximum(
        dinv * (s_ref[:, :] + t1_ref[:, :]) + b1_ref[:, :], 0.0)
    t2_ref[:, :] = dinv * jnp.dot(
        h1, w2_ref[:, :], preferred_element_type=jnp.float32)


def _tc3_body(s_ref, t2_ref, d0_ref, d1_ref, b2_ref, wf_ref, bf_ref,
              x_ref, ws_ref, bs_ref, out_ref):
    dinv = _dinv(d0_ref, d1_ref)
    h2 = jnp.maximum(
        dinv * (s_ref[:, :] + t2_ref[:, :]) + b2_ref[:, :], 0.0)
    out_ref[:, :] = (
        jnp.dot(h2, wf_ref[:, :], preferred_element_type=jnp.float32)
        + bf_ref[:, :]
        + jnp.dot(x_ref[:, :], ws_ref[:, :], preferred_element_type=jnp.float32)
        + bs_ref[:, :])


_feat_spec = pl.BlockSpec((BM, D), lambda i: (i, 0))
_deg_spec = pl.BlockSpec((BM, D), lambda i: (i, 0))
_w_spec = pl.BlockSpec((D, D), lambda i: (0, 0))
_b_spec = pl.BlockSpec((1, D), lambda i: (0, 0))
_GRID = (NP // BM,)
_OUT_F32 = jax.ShapeDtypeStruct((NP, D), jnp.float32)

_tc1 = pl.pallas_call(
    _tc1_body, grid=_GRID,
    in_specs=[_feat_spec, _w_spec, _deg_spec, _deg_spec],
    out_specs=_feat_spec, out_shape=_OUT_F32)

_tc2 = pl.pallas_call(
    _tc2_body, grid=_GRID,
    in_specs=[_feat_spec, _feat_spec, _deg_spec, _deg_spec,
              _b_spec, _w_spec],
    out_specs=_feat_spec, out_shape=_OUT_F32)

_tc3 = pl.pallas_call(
    _tc3_body, grid=_GRID,
    in_specs=[_feat_spec, _feat_spec, _deg_spec, _deg_spec,
              _b_spec, _w_spec, _b_spec, _feat_spec, _w_spec, _b_spec],
    out_specs=_feat_spec, out_shape=_OUT_F32)


def kernel(x, edge_index, W1, b1, W2, b2, W_skip, b_skip, W_final, b_final):
    f32 = jnp.float32
    src = edge_index[0].astype(jnp.int32)
    dst = edge_index[1].astype(jnp.int32)
    pad = EP - E
    src = jnp.concatenate([src, jnp.full((pad,), N, jnp.int32)])
    dst = jnp.concatenate([dst, jnp.full((pad,), N, jnp.int32)])
    xp = jnp.zeros((NP, D), f32).at[:N, :].set(x)
    onesf = jnp.ones((B, D), f32)
    zerosf = jnp.zeros((NP, D), f32)
    b1r = b1.reshape(1, D)
    b2r = b2.reshape(1, D)
    bfr = b_final.reshape(1, D)
    bsr = b_skip.reshape(1, D)

    dd = _deg_kernel(dst, zerosf, onesf)
    d0, d1 = dd[:NP], dd[NP:]
    t1 = _tc1(xp, W1, d0, d1)
    s1 = _edge_kernel(t1, src, dst, zerosf)
    t2 = _tc2(s1, t1, d0, d1, b1r, W2)
    s2 = _edge_kernel(t2, src, dst, zerosf)
    outp = _tc3(s2, t2, d0, d1, b2r, W_final, bfr, xp, W_skip, bsr)
    return outp[:N]
